# baseline (speedup 1.0000x reference)
import dataclasses

import jax
import jax.numpy as jnp
from jax import lax
from jax.experimental import pallas as pl
from jax.experimental.pallas import tpu as pltpu
from jax.experimental.pallas import tpu_sc as plsc

_B = 4096
_F = 26
_V = 100000
_D = 16
_C = 13
_H = 128
_EPS = 1e-5

_NC = 2
_NS = 16
_NW = _NC * _NS
_BROWS = _B // _NW
_CHUNK = 32
_NCHUNK = _BROWS // _CHUNK

_HI = jax.lax.Precision.HIGHEST


_LPW = _B * _F // _NW
_GRP = 16
_NG = _LPW // _GRP
_W = _D // 2


def _gather_body(emb2_hbm, emb1_hbm, idx_hbm, o2_hbm, o1_hbm,
                 idx_v, e2_v, e1_v8, e1c_v, sem2, sem1):
    wid = lax.axis_index("s") * _NC + lax.axis_index("c")
    j0 = wid * _LPW

    pltpu.sync_copy(idx_hbm.at[pl.ds(j0, _LPW)], idx_v)

    @pl.loop(0, _NG)
    def _(g):
        base = g * _GRP
        vreg = idx_v[pl.ds(base, _GRP)]
        for t in range(_GRP):
            k = vreg[t]
            pltpu.make_async_copy(
                emb2_hbm.at[pl.ds(k * _W, _W)],
                e2_v.at[pl.ds((base + t) * _W, _W)],
                sem2,
            ).start()
            k8 = (k >> 3) * 8
            pltpu.make_async_copy(
                emb1_hbm.at[pl.ds(k8, 8)],
                e1_v8.at[pl.ds((base + t) * 8, 8)],
                sem1,
            ).start()

    pltpu.make_async_copy(o2_hbm.at[pl.ds(j0 * _W, _LPW * _W)], e2_v,
                          sem2).wait()
    pltpu.make_async_copy(o2_hbm.at[pl.ds(0, _LPW * 8)], e1_v8, sem1).wait()

    @pl.loop(0, _NG)
    def _(g):
        base = g * _GRP
        vreg = idx_v[pl.ds(base, _GRP)]
        sel = (jax.lax.iota(jnp.int32, _GRP) + base) * 8 + (vreg & 7)
        e1c_v[pl.ds(base, _GRP)] = plsc.load_gather(e1_v8, [sel])

    pltpu.sync_copy(e2_v, o2_hbm.at[pl.ds(j0 * _W, _LPW * _W)])
    pltpu.sync_copy(e1c_v, o1_hbm.at[pl.ds(j0, _LPW)])


def _sc_gather(emb2f, emb1f, idx):
    mesh = plsc.VectorSubcoreMesh(core_axis_name="c", subcore_axis_name="s")
    cp = pltpu.CompilerParams()
    if "needs_layout_passes" in pltpu.CompilerParams.__dataclass_fields__:
        cp = dataclasses.replace(cp, needs_layout_passes=False)
    k = pl.kernel(
        _gather_body,
        out_type=(
            jax.ShapeDtypeStruct((_B * _F * _W,), jnp.float32),
            jax.ShapeDtypeStruct((_B * _F,), jnp.float32),
        ),
        mesh=mesh,
        scratch_types=[
            pltpu.VMEM((_LPW,), jnp.int32),
            pltpu.VMEM((_LPW * _W,), jnp.float32),
            pltpu.VMEM((_LPW * 8,), jnp.float32),
            pltpu.VMEM((_LPW,), jnp.float32),
            pltpu.SemaphoreType.DMA,
            pltpu.SemaphoreType.DMA,
        ],
        compiler_params=cp,
    )
    return k(emb2f, emb1f, idx)


def _bn(x, g, b):
    m = jnp.mean(x, axis=0, keepdims=True)
    xc = x - m
    v = jnp.mean(xc * xc, axis=0, keepdims=True)
    return g * xc / jnp.sqrt(v + _EPS) + b


def _dot(a, b):
    return jax.lax.dot(a, b, precision=_HI, preferred_element_type=jnp.float32)


def _tc_body(dnn0_ref, e1_ref, xi_ref, w1t_ref, b1_ref,
             wdt_ref, bd_ref, gd_ref, bed_ref,
             wat_ref, ba_ref, ga_ref, bea_ref,
             wbt_ref, bb_ref, gb_ref, beb_ref,
             wct_ref, bc_ref, out_ref):
    xi = xi_ref[...]
    dnn0 = dnn0_ref[...].astype(jnp.float32)

    fm1 = jnp.sum(e1_ref[...], axis=1, keepdims=True)
    fm1 = fm1 + _dot(xi, w1t_ref[...]) + b1_ref[...]

    rows = jax.lax.broadcasted_iota(jnp.int32, (_F * _D, _D), 0)
    cols = jax.lax.broadcasted_iota(jnp.int32, (_F * _D, _D), 1)
    sel = (rows % _D == cols).astype(jnp.float32)
    ssum = _dot(dnn0, sel)
    sqsum = _dot(dnn0 * dnn0, sel)
    fm2 = 0.5 * jnp.sum(ssum * ssum - sqsum, axis=1, keepdims=True)

    d = _dot(xi, wdt_ref[...]) + bd_ref[...]
    d = jax.nn.relu(_bn(d, gd_ref[...], bed_ref[...]))
    h = dnn0 + d
    h = _dot(h, wat_ref[...]) + ba_ref[...]
    h = jax.nn.relu(_bn(h, ga_ref[...], bea_ref[...]))
    h = _dot(h, wbt_ref[...]) + bb_ref[...]
    h = jax.nn.relu(_bn(h, gb_ref[...], beb_ref[...]))
    dnn_out = _dot(h, wct_ref[...]) + bc_ref[...]

    out_ref[...] = jax.nn.sigmoid(fm1 + fm2 + dnn_out)


def kernel(xi, xv, W1, b1, emb1, emb2, Wd, bd, gd, bed,
           Wa, ba, ga, bea, Wb, bb, gb, beb, Wc, bc):
    idx = (xv + (jnp.arange(_F, dtype=jnp.int32) * _V)[None, :])
    e2tab = jax.lax.bitcast_convert_type(
        emb2.astype(jnp.bfloat16).reshape(_F * _V * _W, 2), jnp.float32)
    e2f, e1f = _sc_gather(e2tab,
                          emb1[..., 0].reshape(_F * _V),
                          idx.reshape(_B * _F))
    dnn0 = jax.lax.bitcast_convert_type(e2f, jnp.bfloat16)
    dnn0 = dnn0.reshape(_B, _F * _D)
    e1 = e1f.reshape(_B, _F)

    out = pl.pallas_call(
        _tc_body,
        out_shape=jax.ShapeDtypeStruct((_B, 1), jnp.float32),
    )(
        dnn0, e1, xi,
        W1.T, b1.reshape(1, 1),
        Wd.T, bd.reshape(1, -1), gd.reshape(1, -1), bed.reshape(1, -1),
        Wa.T, ba.reshape(1, -1), ga.reshape(1, -1), bea.reshape(1, -1),
        Wb.T, bb.reshape(1, -1), gb.reshape(1, -1), beb.reshape(1, -1),
        Wc.T, bc.reshape(1, 1),
    )
    return out


# speedup vs baseline: 8.6511x; 8.6511x over previous
import dataclasses

import jax
import jax.numpy as jnp
from jax import lax
from jax.experimental import pallas as pl
from jax.experimental.pallas import tpu as pltpu
from jax.experimental.pallas import tpu_sc as plsc

_B = 4096
_F = 26
_V = 100000
_D = 16
_C = 13
_H = 128
_EPS = 1e-5

_NC = 2
_NS = 16
_NW = _NC * _NS
_BROWS = _B // _NW
_CHUNK = 32
_NCHUNK = _BROWS // _CHUNK

_HI = jax.lax.Precision.HIGHEST


_LPW = _B * _F // _NW
_GRP = 16
_NG = _LPW // _GRP


def _gather_body(emb2_hbm, emb1_hbm, idx_hbm, o2_hbm, o1_hbm,
                 idx_v, e2_v, e1_v8, e1c_v, sem2, sem1):
    wid = lax.axis_index("s") * _NC + lax.axis_index("c")
    j0 = wid * _LPW

    pltpu.sync_copy(idx_hbm.at[pl.ds(j0, _LPW)], idx_v)

    @pl.loop(0, _NG)
    def _(g):
        base = g * _GRP
        vreg = idx_v[pl.ds(base, _GRP)]
        for t in range(_GRP):
            k = vreg[t]
            pltpu.make_async_copy(
                emb2_hbm.at[k],
                e2_v.at[pl.ds((base + t) * _D, _D)],
                sem2,
            ).start()
            k8 = (k >> 3) * 8
            pltpu.make_async_copy(
                emb1_hbm.at[pl.ds(k8, 8)],
                e1_v8.at[pl.ds((base + t) * 8, 8)],
                sem1,
            ).start()

    pltpu.make_async_copy(o2_hbm.at[pl.ds(j0 * _D, _LPW * _D)], e2_v,
                          sem2).wait()
    pltpu.make_async_copy(o2_hbm.at[pl.ds(0, _LPW * 8)], e1_v8, sem1).wait()

    @pl.loop(0, _NG)
    def _(g):
        base = g * _GRP
        vreg = idx_v[pl.ds(base, _GRP)]
        sel = (jax.lax.iota(jnp.int32, _GRP) + base) * 8 + (vreg & 7)
        e1c_v[pl.ds(base, _GRP)] = plsc.load_gather(e1_v8, [sel])

    pltpu.sync_copy(e2_v, o2_hbm.at[pl.ds(j0 * _D, _LPW * _D)])
    pltpu.sync_copy(e1c_v, o1_hbm.at[pl.ds(j0, _LPW)])


def _sc_gather(emb2f, emb1f, idx):
    mesh = plsc.VectorSubcoreMesh(core_axis_name="c", subcore_axis_name="s")
    cp = pltpu.CompilerParams(use_tc_tiling_on_sc=False)
    if "needs_layout_passes" in pltpu.CompilerParams.__dataclass_fields__:
        cp = dataclasses.replace(cp, needs_layout_passes=False)
    k = pl.kernel(
        _gather_body,
        out_type=(
            jax.ShapeDtypeStruct((_B * _F * _D,), jnp.float32),
            jax.ShapeDtypeStruct((_B * _F,), jnp.float32),
        ),
        mesh=mesh,
        scratch_types=[
            pltpu.VMEM((_LPW,), jnp.int32),
            pltpu.VMEM((_LPW * _D,), jnp.float32),
            pltpu.VMEM((_LPW * 8,), jnp.float32),
            pltpu.VMEM((_LPW,), jnp.float32),
            pltpu.SemaphoreType.DMA,
            pltpu.SemaphoreType.DMA,
        ],
        compiler_params=cp,
    )
    return k(emb2f, emb1f, idx)


def _bn(x, g, b):
    m = jnp.mean(x, axis=0, keepdims=True)
    xc = x - m
    v = jnp.mean(xc * xc, axis=0, keepdims=True)
    return g * xc / jnp.sqrt(v + _EPS) + b


def _dot(a, b):
    return jax.lax.dot(a, b, precision=_HI, preferred_element_type=jnp.float32)


def _tc_body(dnn0_ref, e1_ref, xi_ref, w1t_ref, b1_ref,
             wdt_ref, bd_ref, gd_ref, bed_ref,
             wat_ref, ba_ref, ga_ref, bea_ref,
             wbt_ref, bb_ref, gb_ref, beb_ref,
             wct_ref, bc_ref, out_ref):
    xi = xi_ref[...]
    dnn0 = dnn0_ref[...]

    fm1 = jnp.sum(e1_ref[...], axis=1, keepdims=True)
    fm1 = fm1 + _dot(xi, w1t_ref[...]) + b1_ref[...]

    rows = jax.lax.broadcasted_iota(jnp.int32, (_F * _D, _D), 0)
    cols = jax.lax.broadcasted_iota(jnp.int32, (_F * _D, _D), 1)
    sel = (rows % _D == cols).astype(jnp.float32)
    ssum = _dot(dnn0, sel)
    sqsum = _dot(dnn0 * dnn0, sel)
    fm2 = 0.5 * jnp.sum(ssum * ssum - sqsum, axis=1, keepdims=True)

    d = _dot(xi, wdt_ref[...]) + bd_ref[...]
    d = jax.nn.relu(_bn(d, gd_ref[...], bed_ref[...]))
    h = dnn0 + d
    h = _dot(h, wat_ref[...]) + ba_ref[...]
    h = jax.nn.relu(_bn(h, ga_ref[...], bea_ref[...]))
    h = _dot(h, wbt_ref[...]) + bb_ref[...]
    h = jax.nn.relu(_bn(h, gb_ref[...], beb_ref[...]))
    dnn_out = _dot(h, wct_ref[...]) + bc_ref[...]

    out_ref[...] = jax.nn.sigmoid(fm1 + fm2 + dnn_out)


def kernel(xi, xv, W1, b1, emb1, emb2, Wd, bd, gd, bed,
           Wa, ba, ga, bea, Wb, bb, gb, beb, Wc, bc):
    idx = (xv + (jnp.arange(_F, dtype=jnp.int32) * _V)[None, :])
    e2f, e1f = _sc_gather(emb2.reshape(_F * _V, _D),
                          emb1[..., 0].reshape(_F * _V),
                          idx.reshape(_B * _F))
    dnn0 = e2f.reshape(_B, _F * _D)
    e1 = e1f.reshape(_B, _F)

    out = pl.pallas_call(
        _tc_body,
        out_shape=jax.ShapeDtypeStruct((_B, 1), jnp.float32),
    )(
        dnn0, e1, xi,
        W1.T, b1.reshape(1, 1),
        Wd.T, bd.reshape(1, -1), gd.reshape(1, -1), bed.reshape(1, -1),
        Wa.T, ba.reshape(1, -1), ga.reshape(1, -1), bea.reshape(1, -1),
        Wb.T, bb.reshape(1, -1), gb.reshape(1, -1), beb.reshape(1, -1),
        Wc.T, bc.reshape(1, 1),
    )
    return out
